# baseline (speedup 1.0000x reference)
import functools

import jax
import jax.numpy as jnp
from jax.experimental import pallas as pl
from jax.experimental.pallas import tpu as pltpu

N = 10000
E = 320000
IN = 128
HID = 64
HEADS = 8
OUT = 16
D = HID * HEADS
SA_HID = 128


def _mm_kernel(a_ref, b_ref, o_ref):
    o_ref[...] = jnp.dot(a_ref[...], b_ref[...],
                         preferred_element_type=jnp.float32)


def _adj_h(adj, h):
    BM = 400
    return pl.pallas_call(
        _mm_kernel,
        grid=(N // BM,),
        in_specs=[
            pl.BlockSpec((BM, N), lambda i: (i, 0)),
            pl.BlockSpec((N, IN), lambda i: (0, 0)),
        ],
        out_specs=pl.BlockSpec((BM, IN), lambda i: (i, 0)),
        out_shape=jax.ShapeDtypeStruct((N, IN), jnp.float32),
    )(adj, h)


def _gat_conv_jax(feat_in, edge_index, W, attn_l, attn_r, bias):
    src = edge_index[0]
    dst = edge_index[1]
    feat = (feat_in @ W).reshape(N, HEADS, HID)
    el = jnp.sum(feat * attn_l[None, :, :], axis=-1)
    er = jnp.sum(feat * attn_r[None, :, :], axis=-1)
    e = jax.nn.leaky_relu(el[src] + er[dst], negative_slope=0.2)
    m = jax.ops.segment_max(e, dst, num_segments=N)
    m = jnp.where(jnp.isfinite(m), m, 0.0)
    ee = jnp.exp(e - m[dst])
    s = jax.ops.segment_sum(ee, dst, num_segments=N)
    alpha = ee / (s[dst] + 1e-9)
    msg = feat[src] * alpha[:, :, None]
    out = jax.ops.segment_sum(msg, dst, num_segments=N)
    out = out + bias.reshape(1, HEADS, HID)
    out = jax.nn.elu(out)
    return out.reshape(N, HEADS * HID)


def kernel(h, adj, edge_index0, edge_index1, fc_W0, attn_l0, attn_r0, bias0,
           fc_W1, attn_l1, attn_r1, bias1, sa_W1, sa_b1, sa_W2, Wp, bp):
    h2 = _adj_h(adj, h)
    z0 = _gat_conv_jax(h2, edge_index0, fc_W0, attn_l0, attn_r0, bias0)
    z1 = _gat_conv_jax(h2, edge_index1, fc_W1, attn_l1, attn_r1, bias1)
    z = jnp.stack([z0, z1], axis=1)
    w = jnp.tanh(z @ sa_W1 + sa_b1) @ sa_W2
    w = w.mean(axis=0)
    beta = jax.nn.softmax(w, axis=0)
    semb = (beta[None, :, :] * z).sum(axis=1)
    return jax.nn.sigmoid(semb @ Wp + bp)


# speedup vs baseline: 15.6380x; 15.6380x over previous
import functools

import jax
import jax.numpy as jnp
from jax import lax
from jax.experimental import pallas as pl
from jax.experimental.pallas import tpu as pltpu
from jax.experimental.pallas import tpu_sc as plsc

N = 10000
E = 320000
IN = 128
HID = 64
HEADS = 8
OUT = 16
D = HID * HEADS
SA_HID = 128

R = 320
NPAD = 32 * R
PADL = 336
SENT = 328
CAP = 20480
CAPA = CAP + 128
EC = 2000
CB = 128
ACC_ROWS = 10368


def _mm_kernel(a_ref, b_ref, o_ref):
    o_ref[...] = jnp.dot(a_ref[...], b_ref[...],
                         preferred_element_type=jnp.float32)


def _adj_h(adj, h):
    BM = 400
    return pl.pallas_call(
        _mm_kernel,
        grid=(N // BM,),
        in_specs=[
            pl.BlockSpec((BM, N), lambda i: (i, 0)),
            pl.BlockSpec((N, IN), lambda i: (0, 0)),
        ],
        out_specs=pl.BlockSpec((BM, IN), lambda i: (i, 0)),
        out_shape=jax.ShapeDtypeStruct((N, IN), jnp.float32),
    )(adj, h)


def _k2_kernel(h2_ref, w_ref, al_ref, ar_ref,
               f0_ref, f1_ref, f2_ref, f3_ref, el_ref, er_ref):
    feat = jnp.dot(h2_ref[...], w_ref[...], preferred_element_type=jnp.float32)
    f0_ref[...] = feat[:, 0:128]
    f1_ref[...] = feat[:, 128:256]
    f2_ref[...] = feat[:, 256:384]
    f3_ref[...] = feat[:, 384:512]
    el_ref[...] = jnp.dot(feat, al_ref[...], preferred_element_type=jnp.float32,
                          precision=lax.Precision.HIGHEST)
    er_ref[...] = jnp.dot(feat, ar_ref[...], preferred_element_type=jnp.float32,
                          precision=lax.Precision.HIGHEST)


def _feat_el_er(h2, W, AL, AR):
    BN = 400
    return pl.pallas_call(
        _k2_kernel,
        grid=(N // BN,),
        in_specs=[
            pl.BlockSpec((BN, IN), lambda i: (i, 0)),
            pl.BlockSpec((IN, D), lambda i: (0, 0)),
            pl.BlockSpec((D, HEADS), lambda i: (0, 0)),
            pl.BlockSpec((D, HEADS), lambda i: (0, 0)),
        ],
        out_specs=[pl.BlockSpec((BN, 128), lambda i: (i, 0))] * 4
        + [pl.BlockSpec((BN, HEADS), lambda i: (i, 0))] * 2,
        out_shape=[jax.ShapeDtypeStruct((N, 128), jnp.float32)] * 4
        + [jax.ShapeDtypeStruct((N, HEADS), jnp.float32)] * 2,
    )(h2, W, AL, AR)


def _make_edge_phase():
    mesh = plsc.VectorSubcoreMesh(core_axis_name="c", subcore_axis_name="s")
    out_type = [
        jax.ShapeDtypeStruct((32 * CAP,), jnp.int32),
        jax.ShapeDtypeStruct((32 * CAP,), jnp.int32),
        jax.ShapeDtypeStruct((HEADS * 32 * CAP,), jnp.float32),
        jax.ShapeDtypeStruct((HEADS * NPAD,), jnp.float32),
        jax.ShapeDtypeStruct((32 * 16,), jnp.int32),
    ]
    scratch = [
        pltpu.VMEM((EC,), jnp.int32),
        pltpu.VMEM((EC,), jnp.int32),
        pltpu.VMEM((CAPA,), jnp.int32),
        pltpu.VMEM((CAPA,), jnp.int32),
        pltpu.VMEM((CAP,), jnp.float32),
        pltpu.VMEM((10256,), jnp.float32),
        pltpu.VMEM((10256,), jnp.float32),
        pltpu.VMEM((16 * PADL,), jnp.float32),
        pltpu.VMEM((16 * PADL,), jnp.float32),
        pltpu.VMEM((PADL,), jnp.float32),
        pltpu.VMEM((PADL,), jnp.float32),
        pltpu.VMEM((16,), jnp.int32),
    ]

    @functools.partial(pl.kernel, out_type=out_type, mesh=mesh,
                       scratch_types=scratch,
                       compiler_params=pltpu.CompilerParams(
                           needs_layout_passes=False))
    def k(src_h, dst_h, el_t, er_t, srcc, dstlc, eec, s_t, counts,
          srcb, dstb, srcl, dstl, ebuf, elb, erb, mtab, stab, mred, sred,
          cbuf):
        c = lax.axis_index("c")
        s = lax.axis_index("s")
        wid = s * 2 + c
        lo = wid * R
        iota = lax.broadcasted_iota(jnp.int32, (16,), 0)
        zf16 = jnp.zeros((16,), jnp.float32)
        lo16 = jnp.full((16,), lo, jnp.int32)

        def chunk_body(ci, ptr):
            pltpu.sync_copy(src_h.at[pl.ds(ci * EC, EC)], srcb)
            pltpu.sync_copy(dst_h.at[pl.ds(ci * EC, EC)], dstb)

            def vbody(i, ptr):
                sv = srcb[pl.ds(i * 16, 16)]
                dv = dstb[pl.ds(i * 16, 16)]
                dl = dv - lo16
                mask = (dl >= 0) & (dl < R)
                cum = plsc.cumsum(mask.astype(jnp.int32))
                pos = jnp.full((16,), ptr, jnp.int32) + cum - 1
                plsc.store_scatter(srcl, [pos], sv, mask=mask)
                plsc.store_scatter(dstl, [pos], dl, mask=mask)
                return ptr + cum[15]

            return lax.fori_loop(0, EC // 16, vbody, ptr)

        ptr = lax.fori_loop(0, E // EC, chunk_body, jnp.int32(0))

        tgt = ((ptr + CB - 1) // CB) * CB
        sent16 = jnp.full((16,), SENT, jnp.int32)
        zi16 = jnp.zeros((16,), jnp.int32)
        tgt16 = jnp.full((16,), tgt, jnp.int32)
        ptr16 = jnp.full((16,), ptr, jnp.int32)
        for kp in range(CB // 16):
            idx = ptr16 + kp * 16 + iota
            pm = idx < tgt16
            plsc.store_scatter(srcl, [idx], zi16, mask=pm)
            plsc.store_scatter(dstl, [idx], sent16, mask=pm)

        def wlist(kk, _):
            sl = pl.ds(kk * 2048, 2048)
            slh = pl.ds(wid * CAP + kk * 2048, 2048)
            pltpu.sync_copy(srcl.at[sl], srcc.at[slh])
            pltpu.sync_copy(dstl.at[sl], dstlc.at[slh])
            return _

        lax.fori_loop(0, (tgt + 2047) // 2048, wlist, 0)
        cbuf[...] = jnp.full((16,), tgt, jnp.int32)
        pltpu.sync_copy(cbuf, counts.at[pl.ds(wid * 16, 16)])

        nv = tgt // 16

        for h in range(HEADS):
            def initt(i, _):
                mtab[pl.ds(i * 16, 16)] = jnp.full((16,), -1e30, jnp.float32)
                stab[pl.ds(i * 16, 16)] = zf16
                return _

            lax.fori_loop(0, 16 * PADL // 16, initt, 0)

            pltpu.sync_copy(el_t.at[pl.ds(h * N, N)], elb.at[pl.ds(0, N)])
            pltpu.sync_copy(er_t.at[pl.ds(h * N, N)], erb.at[pl.ds(0, N)])
            for z0 in range(N, 10256, 16):
                elb[pl.ds(z0, 16)] = zf16
                erb[pl.ds(z0, 16)] = zf16

            def p1(i, _):
                sv = srcl[pl.ds(i * 16, 16)]
                dl = dstl[pl.ds(i * 16, 16)]
                x = plsc.load_gather(elb, [sv]) + plsc.load_gather(erb, [dl + lo16])
                e = jnp.maximum(x, 0.2 * x)
                ebuf[pl.ds(i * 16, 16)] = e
                midx = iota * PADL + dl
                cur = plsc.load_gather(mtab, [midx])
                plsc.store_scatter(mtab, [midx], jnp.maximum(cur, e))
                return _

            lax.fori_loop(0, nv, p1, 0)

            def mr(cc, _):
                v = mtab[pl.ds(cc * 16, 16)]
                for l in range(1, 16):
                    v = jnp.maximum(v, mtab[pl.ds(l * PADL + cc * 16, 16)])
                mred[pl.ds(cc * 16, 16)] = v
                return _

            lax.fori_loop(0, PADL // 16, mr, 0)

            def p2(i, _):
                dl = dstl[pl.ds(i * 16, 16)]
                e = ebuf[pl.ds(i * 16, 16)]
                ee = jnp.exp(e - plsc.load_gather(mred, [dl]))
                ee = jnp.where(dl < R, ee, zf16)
                ebuf[pl.ds(i * 16, 16)] = ee
                sidx = iota * PADL + dl
                cur = plsc.load_gather(stab, [sidx])
                plsc.store_scatter(stab, [sidx], cur + ee)
                return _

            lax.fori_loop(0, nv, p2, 0)

            def sr(cc, _):
                v = stab[pl.ds(cc * 16, 16)]
                for l in range(1, 16):
                    v = v + stab[pl.ds(l * PADL + cc * 16, 16)]
                sred[pl.ds(cc * 16, 16)] = v
                return _

            lax.fori_loop(0, PADL // 16, sr, 0)

            pltpu.sync_copy(sred.at[pl.ds(0, R)],
                            s_t.at[pl.ds(h * NPAD + lo, R)])

            def wee(kk, _):
                sl = pl.ds(kk * 2048, 2048)
                pltpu.sync_copy(
                    ebuf.at[sl],
                    eec.at[pl.ds((h * 32 + wid) * CAP + kk * 2048, 2048)])
                return _

            lax.fori_loop(0, (tgt + 2047) // 2048, wee, 0)

    return k


def _make_aggregate(p):
    mesh = plsc.VectorSubcoreMesh(core_axis_name="c", subcore_axis_name="s")
    out_type = jax.ShapeDtypeStruct((2, ACC_ROWS, 128), jnp.float32)
    scratch = [
        pltpu.VMEM((CB, 128), jnp.float32),
        pltpu.VMEM((CB,), jnp.int32),
        pltpu.VMEM((CB,), jnp.int32),
        pltpu.VMEM((CB,), jnp.int32),
        pltpu.VMEM((CB + 16,), jnp.float32),
        pltpu.VMEM((CB + 16,), jnp.float32),
        pltpu.VMEM((512,), jnp.int32),
        pltpu.VMEM_SHARED((ACC_ROWS, 128), jnp.float32),
        pltpu.SemaphoreType.DMA,
    ]

    @functools.partial(pl.kernel, out_type=out_type, mesh=mesh,
                       scratch_types=scratch,
                       compiler_params=pltpu.CompilerParams(
                           needs_layout_passes=False))
    def k(feata, featb, srcc, dstlc, eec, counts, out,
          gath, sidx, dlb, gidx, ee0, ee1, cnt, accum, sem):
        c = lax.axis_index("c")
        s = lax.axis_index("s")
        zf16 = jnp.zeros((16,), jnp.float32)

        def zrow(i, _):
            for q in range(8):
                gath[i, pl.ds(q * 16, 16)] = zf16
            return _

        lax.fori_loop(0, CB, zrow, 0)
        base = s * (ACC_ROWS // 16)
        for rr in range(5):
            pltpu.sync_copy(gath, accum.at[pl.ds(base + rr * 128, 128)])
        pltpu.sync_copy(gath.at[pl.ds(0, 8)], accum.at[pl.ds(base + 640, 8)])
        plsc.subcore_barrier()

        pltpu.sync_copy(counts, cnt)

        def work(feat, gg):
            h0 = 2 * gg
            h1 = 2 * gg + 1
            for jj in range(2):
                j = s * 2 + jj
                npad = cnt[pl.ds(j * 16, 16)][0]

                def chunk(cc2, _):
                    c0 = cc2 * CB
                    pltpu.sync_copy(srcc.at[pl.ds(j * CAP + c0, CB)], sidx)
                    pltpu.sync_copy(dstlc.at[pl.ds(j * CAP + c0, CB)], dlb)
                    pltpu.sync_copy(
                        eec.at[pl.ds((h0 * 32 + j) * CAP + c0, CB)],
                        ee0.at[pl.ds(0, CB)])
                    pltpu.sync_copy(
                        eec.at[pl.ds((h1 * 32 + j) * CAP + c0, CB)],
                        ee1.at[pl.ds(0, CB)])
                    pltpu.async_copy(feat.at[sidx], gath, sem).wait()
                    base16 = jnp.full((16,), j * R, jnp.int32)
                    for kk in range(CB // 16):
                        sl = pl.ds(kk * 16, 16)
                        gidx[sl] = dlb[sl] + base16

                    def scale(e2, _2):
                        a0 = jnp.full((16,), ee0[pl.ds(e2, 16)][0],
                                      jnp.float32)
                        a1 = jnp.full((16,), ee1[pl.ds(e2, 16)][0],
                                      jnp.float32)
                        for q in range(4):
                            sl = pl.ds(q * 16, 16)
                            gath[e2, sl] = gath[e2, sl] * a0
                        for q in range(4, 8):
                            sl = pl.ds(q * 16, 16)
                            gath[e2, sl] = gath[e2, sl] * a1
                        return _2

                    lax.fori_loop(0, CB, scale, 0)
                    pltpu.sync_copy(gath, accum.at[gidx], add=True)
                    return _

                lax.fori_loop(0, npad // CB, chunk, 0)

        @pl.when(c == 0)
        def _():
            work(feata, 2 * p + 0)

        @pl.when(c == 1)
        def _():
            work(featb, 2 * p + 1)

        plsc.subcore_barrier()
        for rr in range(5):
            sl = pl.ds(base + rr * 128, 128)
            pltpu.sync_copy(accum.at[sl], out.at[c, sl])
        sl = pl.ds(base + 640, 8)
        pltpu.sync_copy(accum.at[sl], out.at[c, sl])

    return k


BN3 = 400


def _k3_kernel(a01_ref, a23_ref, s_ref, bias_ref, w1_ref, b1_ref, wp_ref,
               y_ref, ts_ref):
    zs = []
    for g in range(4):
        acc = (a01_ref if g < 2 else a23_ref)[g % 2]
        d0 = s_ref[:, 2 * g]
        d1 = s_ref[:, 2 * g + 1]
        div = jnp.concatenate(
            [jnp.broadcast_to(d0[:, None], (BN3, 64)),
             jnp.broadcast_to(d1[:, None], (BN3, 64))], axis=1)
        zs.append(acc / (div + 1e-9) + bias_ref[0:1, g * 128:(g + 1) * 128])
    zc = jnp.concatenate(zs, axis=1)
    z = jnp.where(zc > 0, zc, jnp.exp(jnp.minimum(zc, 0.0)) - 1.0)
    y_ref[...] = jnp.dot(z, wp_ref[...], preferred_element_type=jnp.float32)
    t = jnp.tanh(jnp.dot(z, w1_ref[...], preferred_element_type=jnp.float32)
                 + b1_ref[0:1, :])
    tsum = jnp.broadcast_to(jnp.sum(t, axis=0, keepdims=True), (8, SA_HID))

    @pl.when(pl.program_id(0) == 0)
    def _():
        ts_ref[...] = tsum

    @pl.when(pl.program_id(0) > 0)
    def _():
        ts_ref[...] = ts_ref[...] + tsum


def _k3(acc01, acc23, s_nh, bias8, w1, b18, wp):
    return pl.pallas_call(
        _k3_kernel,
        grid=(N // BN3,),
        in_specs=[
            pl.BlockSpec((2, BN3, 128), lambda i: (0, i, 0)),
            pl.BlockSpec((2, BN3, 128), lambda i: (0, i, 0)),
            pl.BlockSpec((BN3, HEADS), lambda i: (i, 0)),
            pl.BlockSpec((8, D), lambda i: (0, 0)),
            pl.BlockSpec((D, SA_HID), lambda i: (0, 0)),
            pl.BlockSpec((8, SA_HID), lambda i: (0, 0)),
            pl.BlockSpec((D, OUT), lambda i: (0, 0)),
        ],
        out_specs=[
            pl.BlockSpec((BN3, OUT), lambda i: (i, 0)),
            pl.BlockSpec((8, SA_HID), lambda i: (0, 0)),
        ],
        out_shape=[
            jax.ShapeDtypeStruct((N, OUT), jnp.float32),
            jax.ShapeDtypeStruct((8, SA_HID), jnp.float32),
        ],
    )(acc01, acc23, s_nh, bias8, w1, b18, wp)


def _k4_kernel(y0_ref, y1_ref, bb_ref, bp_ref, o_ref):
    o_ref[...] = jax.nn.sigmoid(
        y0_ref[...] * bb_ref[0] + y1_ref[...] * bb_ref[1] + bp_ref[0:1, :])


def _k4(y0, y1, bb, bp8):
    return pl.pallas_call(
        _k4_kernel,
        grid=(1,),
        in_specs=[
            pl.BlockSpec((N, OUT), lambda i: (0, 0)),
            pl.BlockSpec((N, OUT), lambda i: (0, 0)),
            pl.BlockSpec(memory_space=pltpu.SMEM),
            pl.BlockSpec((8, OUT), lambda i: (0, 0)),
        ],
        out_specs=pl.BlockSpec((N, OUT), lambda i: (0, 0)),
        out_shape=jax.ShapeDtypeStruct((N, OUT), jnp.float32),
    )(y0, y1, bb, bp8)


def _attn_mat(a):
    eye = jnp.eye(HEADS, dtype=jnp.float32)
    return (eye[:, None, :] * a[:, :, None]).reshape(D, HEADS)


def kernel(h, adj, edge_index0, edge_index1, fc_W0, attn_l0, attn_r0, bias0,
           fc_W1, attn_l1, attn_r1, bias1, sa_W1, sa_b1, sa_W2, Wp, bp):
    h2 = _adj_h(adj, h)

    edge_phase = _make_edge_phase()
    agg0 = _make_aggregate(0)
    agg1 = _make_aggregate(1)

    ys = []
    ws = []
    for ei, W, al, ar, bias in (
            (edge_index0, fc_W0, attn_l0, attn_r0, bias0),
            (edge_index1, fc_W1, attn_l1, attn_r1, bias1)):
        f0, f1, f2, f3, el, er = _feat_el_er(h2, W, _attn_mat(al),
                                             _attn_mat(ar))
        el_t = el.T.reshape(-1)
        er_t = er.T.reshape(-1)
        srcc, dstlc, eec, s_t, counts = edge_phase(ei[0], ei[1], el_t, er_t)
        acc01 = agg0(f0, f1, srcc, dstlc, eec, counts)
        acc23 = agg1(f2, f3, srcc, dstlc, eec, counts)
        s_nh = s_t.reshape(HEADS, NPAD)[:, :N].T
        bias8 = jnp.broadcast_to(bias[None, :], (8, D))
        b18 = jnp.broadcast_to(sa_b1[None, :], (8, SA_HID))
        y, ts = _k3(acc01, acc23, s_nh, bias8, sa_W1, b18, Wp)
        ys.append(y)
        ws.append(jnp.dot(ts[0], sa_W2[:, 0]) / N)

    beta = jax.nn.softmax(jnp.stack(ws))
    bp8 = jnp.broadcast_to(bp[None, :], (8, OUT))
    return _k4(ys[0], ys[1], beta, bp8)


# speedup vs baseline: 19.2691x; 1.2322x over previous
import functools

import jax
import jax.numpy as jnp
from jax import lax
from jax.experimental import pallas as pl
from jax.experimental.pallas import tpu as pltpu
from jax.experimental.pallas import tpu_sc as plsc

N = 10000
E = 320000
IN = 128
HID = 64
HEADS = 8
OUT = 16
D = HID * HEADS
SA_HID = 128

R = 320
NPAD = 32 * R
PADL = 336
SENT = 328
CAP = 20480
CAPA = CAP + 128
EC = 2000
CB = 128
ACC_ROWS = 10368


def _mm_kernel(a_ref, b_ref, o_ref):
    o_ref[...] = jnp.dot(a_ref[...], b_ref[...],
                         preferred_element_type=jnp.float32)


def _adj_h(adj, h):
    BM = 400
    return pl.pallas_call(
        _mm_kernel,
        grid=(N // BM,),
        in_specs=[
            pl.BlockSpec((BM, N), lambda i: (i, 0)),
            pl.BlockSpec((N, IN), lambda i: (0, 0)),
        ],
        out_specs=pl.BlockSpec((BM, IN), lambda i: (i, 0)),
        out_shape=jax.ShapeDtypeStruct((N, IN), jnp.float32),
    )(adj, h)


def _k2_kernel(h2_ref, w_ref, al_ref, ar_ref,
               f0_ref, f1_ref, f2_ref, f3_ref, el_ref, er_ref):
    feat = jnp.dot(h2_ref[...], w_ref[...], preferred_element_type=jnp.float32)
    f0_ref[...] = feat[:, 0:128]
    f1_ref[...] = feat[:, 128:256]
    f2_ref[...] = feat[:, 256:384]
    f3_ref[...] = feat[:, 384:512]
    el_ref[...] = jnp.dot(feat, al_ref[...], preferred_element_type=jnp.float32,
                          precision=lax.Precision.HIGHEST)
    er_ref[...] = jnp.dot(feat, ar_ref[...], preferred_element_type=jnp.float32,
                          precision=lax.Precision.HIGHEST)


def _feat_el_er(h2, W, AL, AR):
    BN = 400
    return pl.pallas_call(
        _k2_kernel,
        grid=(N // BN,),
        in_specs=[
            pl.BlockSpec((BN, IN), lambda i: (i, 0)),
            pl.BlockSpec((IN, D), lambda i: (0, 0)),
            pl.BlockSpec((D, HEADS), lambda i: (0, 0)),
            pl.BlockSpec((D, HEADS), lambda i: (0, 0)),
        ],
        out_specs=[pl.BlockSpec((BN, 128), lambda i: (i, 0))] * 4
        + [pl.BlockSpec((BN, HEADS), lambda i: (i, 0))] * 2,
        out_shape=[jax.ShapeDtypeStruct((N, 128), jnp.float32)] * 4
        + [jax.ShapeDtypeStruct((N, HEADS), jnp.float32)] * 2,
    )(h2, W, AL, AR)


def _make_edge_phase():
    mesh = plsc.VectorSubcoreMesh(core_axis_name="c", subcore_axis_name="s")
    out_type = [
        jax.ShapeDtypeStruct((32 * CAP,), jnp.int32),
        jax.ShapeDtypeStruct((32 * CAP,), jnp.int32),
        jax.ShapeDtypeStruct((HEADS * 32 * CAP,), jnp.float32),
        jax.ShapeDtypeStruct((HEADS * NPAD,), jnp.float32),
        jax.ShapeDtypeStruct((32 * 16,), jnp.int32),
    ]
    scratch = [
        pltpu.VMEM((EC,), jnp.int32),
        pltpu.VMEM((EC,), jnp.int32),
        pltpu.VMEM((EC,), jnp.int32),
        pltpu.VMEM((EC,), jnp.int32),
        pltpu.SemaphoreType.DMA,
        pltpu.SemaphoreType.DMA,
        pltpu.VMEM((CAPA,), jnp.int32),
        pltpu.VMEM((CAPA,), jnp.int32),
        pltpu.VMEM((CAP,), jnp.float32),
        pltpu.VMEM((10256,), jnp.float32),
        pltpu.VMEM((10256,), jnp.float32),
        pltpu.VMEM((16 * PADL,), jnp.float32),
        pltpu.VMEM((16 * PADL,), jnp.float32),
        pltpu.VMEM((PADL,), jnp.float32),
        pltpu.VMEM((PADL,), jnp.float32),
        pltpu.VMEM((16,), jnp.int32),
    ]

    @functools.partial(pl.kernel, out_type=out_type, mesh=mesh,
                       scratch_types=scratch,
                       compiler_params=pltpu.CompilerParams(
                           needs_layout_passes=False))
    def k(src_h, dst_h, el_t, er_t, srcc, dstlc, eec, s_t, counts,
          srcb, dstb, srcb2, dstb2, semA, semB,
          srcl, dstl, ebuf, elb, erb, mtab, stab, mred, sred,
          cbuf):
        c = lax.axis_index("c")
        s = lax.axis_index("s")
        wid = s * 2 + c
        lo = wid * R
        iota = lax.broadcasted_iota(jnp.int32, (16,), 0)
        zf16 = jnp.zeros((16,), jnp.float32)
        lo16 = jnp.full((16,), lo, jnp.int32)

        def issue(ci, sb, db, sema, semb):
            pltpu.async_copy(src_h.at[pl.ds(ci * EC, EC)], sb, sema)
            pltpu.async_copy(dst_h.at[pl.ds(ci * EC, EC)], db, semb)

        def wait(ci, sb, db, sema, semb):
            pltpu.make_async_copy(src_h.at[pl.ds(ci * EC, EC)], sb, sema).wait()
            pltpu.make_async_copy(dst_h.at[pl.ds(ci * EC, EC)], db, semb).wait()

        def scan(sb, db, ptr):
            def vbody(i, ptr):
                sv = sb[pl.ds(i * 16, 16)]
                dv = db[pl.ds(i * 16, 16)]
                dl = dv - lo16
                mask = (dl >= 0) & (dl < R)
                cum = plsc.cumsum(mask.astype(jnp.int32))
                pos = jnp.full((16,), ptr, jnp.int32) + cum - 1
                plsc.store_scatter(srcl, [pos], sv, mask=mask)
                plsc.store_scatter(dstl, [pos], dl, mask=mask)
                return ptr + cum[15]

            return lax.fori_loop(0, EC // 16, vbody, ptr)

        NCH = E // EC

        issue(0, srcb, dstb, semA, semB)

        def chunk_pair(kk, ptr):
            ci = kk * 2
            issue(ci + 1, srcb2, dstb2, semA, semB)
            wait(ci, srcb, dstb, semA, semB)
            ptr = scan(srcb, dstb, ptr)

            @pl.when(kk + 1 < NCH // 2)
            def _issue_next():
                issue(ci + 2, srcb, dstb, semA, semB)

            wait(ci + 1, srcb2, dstb2, semA, semB)
            return scan(srcb2, dstb2, ptr)

        ptr = lax.fori_loop(0, NCH // 2, chunk_pair, jnp.int32(0))

        tgt = ((ptr + 2 * CB - 1) // (2 * CB)) * (2 * CB)
        sent16 = jnp.full((16,), SENT, jnp.int32)
        zi16 = jnp.zeros((16,), jnp.int32)
        tgt16 = jnp.full((16,), tgt, jnp.int32)
        ptr16 = jnp.full((16,), ptr, jnp.int32)
        for kp in range(2 * CB // 16):
            idx = ptr16 + kp * 16 + iota
            pm = idx < tgt16
            plsc.store_scatter(srcl, [idx], zi16, mask=pm)
            plsc.store_scatter(dstl, [idx], sent16, mask=pm)

        def wlist(kk, _):
            sl = pl.ds(kk * 2048, 2048)
            slh = pl.ds(wid * CAP + kk * 2048, 2048)
            pltpu.sync_copy(srcl.at[sl], srcc.at[slh])
            pltpu.sync_copy(dstl.at[sl], dstlc.at[slh])
            return _

        lax.fori_loop(0, (tgt + 2047) // 2048, wlist, 0)
        cbuf[...] = jnp.full((16,), tgt, jnp.int32)
        pltpu.sync_copy(cbuf, counts.at[pl.ds(wid * 16, 16)])

        nv = tgt // 16

        for h in range(HEADS):
            def initt(i, _):
                mtab[pl.ds(i * 16, 16)] = jnp.full((16,), -1e30, jnp.float32)
                stab[pl.ds(i * 16, 16)] = zf16
                return _

            lax.fori_loop(0, 16 * PADL // 16, initt, 0)

            pltpu.sync_copy(el_t.at[pl.ds(h * N, N)], elb.at[pl.ds(0, N)])
            pltpu.sync_copy(er_t.at[pl.ds(h * N, N)], erb.at[pl.ds(0, N)])
            for z0 in range(N, 10256, 16):
                elb[pl.ds(z0, 16)] = zf16
                erb[pl.ds(z0, 16)] = zf16

            def p1(i, _):
                sv = srcl[pl.ds(i * 16, 16)]
                dl = dstl[pl.ds(i * 16, 16)]
                x = plsc.load_gather(elb, [sv]) + plsc.load_gather(erb, [dl + lo16])
                e = jnp.maximum(x, 0.2 * x)
                ebuf[pl.ds(i * 16, 16)] = e
                midx = iota * PADL + dl
                cur = plsc.load_gather(mtab, [midx])
                plsc.store_scatter(mtab, [midx], jnp.maximum(cur, e))
                return _

            lax.fori_loop(0, nv, p1, 0)

            def mr(cc, _):
                v = mtab[pl.ds(cc * 16, 16)]
                for l in range(1, 16):
                    v = jnp.maximum(v, mtab[pl.ds(l * PADL + cc * 16, 16)])
                mred[pl.ds(cc * 16, 16)] = v
                return _

            lax.fori_loop(0, PADL // 16, mr, 0)

            def p2(i, _):
                dl = dstl[pl.ds(i * 16, 16)]
                e = ebuf[pl.ds(i * 16, 16)]
                ee = jnp.exp(e - plsc.load_gather(mred, [dl]))
                ee = jnp.where(dl < R, ee, zf16)
                ebuf[pl.ds(i * 16, 16)] = ee
                sidx = iota * PADL + dl
                cur = plsc.load_gather(stab, [sidx])
                plsc.store_scatter(stab, [sidx], cur + ee)
                return _

            lax.fori_loop(0, nv, p2, 0)

            def sr(cc, _):
                v = stab[pl.ds(cc * 16, 16)]
                for l in range(1, 16):
                    v = v + stab[pl.ds(l * PADL + cc * 16, 16)]
                sred[pl.ds(cc * 16, 16)] = v
                return _

            lax.fori_loop(0, PADL // 16, sr, 0)

            pltpu.sync_copy(sred.at[pl.ds(0, R)],
                            s_t.at[pl.ds(h * NPAD + lo, R)])

            def wee(kk, _):
                sl = pl.ds(kk * 2048, 2048)
                pltpu.sync_copy(
                    ebuf.at[sl],
                    eec.at[pl.ds((h * 32 + wid) * CAP + kk * 2048, 2048)])
                return _

            lax.fori_loop(0, (tgt + 2047) // 2048, wee, 0)

    return k


def _make_aggregate(p):
    mesh = plsc.VectorSubcoreMesh(core_axis_name="c", subcore_axis_name="s")
    out_type = jax.ShapeDtypeStruct((2, ACC_ROWS, 128), jnp.float32)
    scratch = [
        pltpu.VMEM((CB, 128), jnp.float32),
        pltpu.VMEM((CB, 128), jnp.float32),
        pltpu.VMEM((CB,), jnp.int32),
        pltpu.VMEM((CB,), jnp.int32),
        pltpu.VMEM((CB,), jnp.int32),
        pltpu.VMEM((CB,), jnp.int32),
        pltpu.VMEM((CB,), jnp.int32),
        pltpu.VMEM((CB + 16,), jnp.float32),
        pltpu.VMEM((CB + 16,), jnp.float32),
        pltpu.VMEM((CB + 16,), jnp.float32),
        pltpu.VMEM((CB + 16,), jnp.float32),
        pltpu.VMEM((512,), jnp.int32),
        pltpu.VMEM_SHARED((ACC_ROWS, 128), jnp.float32),
        pltpu.SemaphoreType.DMA,
        pltpu.SemaphoreType.DMA,
    ]

    @functools.partial(pl.kernel, out_type=out_type, mesh=mesh,
                       scratch_types=scratch,
                       compiler_params=pltpu.CompilerParams(
                           needs_layout_passes=False))
    def k(feata, featb, srcc, dstlc, eec, counts, out,
          gath0, gath1, sidx0, sidx1, dlb0, dlb1, gidx,
          ee0a, ee1a, ee0b, ee1b, cnt, accum, sem0, sem1):
        c = lax.axis_index("c")
        s = lax.axis_index("s")
        zf16 = jnp.zeros((16,), jnp.float32)

        def zrow(i, _):
            for q in range(8):
                gath0[i, pl.ds(q * 16, 16)] = zf16
            return _

        lax.fori_loop(0, CB, zrow, 0)
        base = s * (ACC_ROWS // 16)
        for rr in range(5):
            pltpu.sync_copy(gath0, accum.at[pl.ds(base + rr * 128, 128)])
        pltpu.sync_copy(gath0.at[pl.ds(0, 8)], accum.at[pl.ds(base + 640, 8)])
        plsc.subcore_barrier()

        pltpu.sync_copy(counts, cnt)

        def work(feat, gg):
            h0 = 2 * gg
            h1 = 2 * gg + 1

            def loadlists(j, c0, sidx, dlb, ee0, ee1):
                pltpu.sync_copy(srcc.at[pl.ds(j * CAP + c0, CB)], sidx)
                pltpu.sync_copy(dstlc.at[pl.ds(j * CAP + c0, CB)], dlb)
                pltpu.sync_copy(
                    eec.at[pl.ds((h0 * 32 + j) * CAP + c0, CB)],
                    ee0.at[pl.ds(0, CB)])
                pltpu.sync_copy(
                    eec.at[pl.ds((h1 * 32 + j) * CAP + c0, CB)],
                    ee1.at[pl.ds(0, CB)])

            def process(j, gath, dlb, ee0, ee1):
                base16 = jnp.full((16,), j * R, jnp.int32)
                for kk in range(CB // 16):
                    sl = pl.ds(kk * 16, 16)
                    gidx[sl] = dlb[sl] + base16

                def scale(e2, _2):
                    a0 = jnp.full((16,), ee0[pl.ds(e2, 16)][0], jnp.float32)
                    a1 = jnp.full((16,), ee1[pl.ds(e2, 16)][0], jnp.float32)
                    for q in range(4):
                        sl = pl.ds(q * 16, 16)
                        gath[e2, sl] = gath[e2, sl] * a0
                    for q in range(4, 8):
                        sl = pl.ds(q * 16, 16)
                        gath[e2, sl] = gath[e2, sl] * a1
                    return _2

                lax.fori_loop(0, CB, scale, 0)
                pltpu.sync_copy(gath, accum.at[gidx], add=True)

            for jj in range(2):
                j = s * 2 + jj
                npad = cnt[pl.ds(j * 16, 16)][0]
                npairs = npad // (2 * CB)

                @pl.when(npairs > 0)
                def _():
                    loadlists(j, 0, sidx0, dlb0, ee0a, ee1a)
                    pltpu.async_copy(feat.at[sidx0], gath0, sem0)

                def pair(kk, carry):
                    c0 = kk * 2 * CB
                    loadlists(j, c0 + CB, sidx1, dlb1, ee0b, ee1b)
                    pltpu.async_copy(feat.at[sidx1], gath1, sem1)
                    pltpu.make_async_copy(feat.at[sidx0], gath0, sem0).wait()
                    process(j, gath0, dlb0, ee0a, ee1a)

                    @pl.when(kk + 1 < npairs)
                    def _issue_next():
                        loadlists(j, c0 + 2 * CB, sidx0, dlb0, ee0a, ee1a)
                        pltpu.async_copy(feat.at[sidx0], gath0, sem0)

                    pltpu.make_async_copy(feat.at[sidx1], gath1, sem1).wait()
                    process(j, gath1, dlb1, ee0b, ee1b)
                    return carry

                lax.fori_loop(0, npairs, pair, 0)

        @pl.when(c == 0)
        def _():
            work(feata, 2 * p + 0)

        @pl.when(c == 1)
        def _():
            work(featb, 2 * p + 1)

        plsc.subcore_barrier()
        for rr in range(5):
            sl = pl.ds(base + rr * 128, 128)
            pltpu.sync_copy(accum.at[sl], out.at[c, sl])
        sl = pl.ds(base + 640, 8)
        pltpu.sync_copy(accum.at[sl], out.at[c, sl])

    return k


BN3 = 400


def _k3_kernel(a01_ref, a23_ref, s_ref, bias_ref, w1_ref, b1_ref, wp_ref,
               y_ref, ts_ref):
    zs = []
    for g in range(4):
        acc = (a01_ref if g < 2 else a23_ref)[g % 2]
        d0 = s_ref[:, 2 * g]
        d1 = s_ref[:, 2 * g + 1]
        div = jnp.concatenate(
            [jnp.broadcast_to(d0[:, None], (BN3, 64)),
             jnp.broadcast_to(d1[:, None], (BN3, 64))], axis=1)
        zs.append(acc / (div + 1e-9) + bias_ref[0:1, g * 128:(g + 1) * 128])
    zc = jnp.concatenate(zs, axis=1)
    z = jnp.where(zc > 0, zc, jnp.exp(jnp.minimum(zc, 0.0)) - 1.0)
    y_ref[...] = jnp.dot(z, wp_ref[...], preferred_element_type=jnp.float32)
    t = jnp.tanh(jnp.dot(z, w1_ref[...], preferred_element_type=jnp.float32)
                 + b1_ref[0:1, :])
    tsum = jnp.broadcast_to(jnp.sum(t, axis=0, keepdims=True), (8, SA_HID))

    @pl.when(pl.program_id(0) == 0)
    def _():
        ts_ref[...] = tsum

    @pl.when(pl.program_id(0) > 0)
    def _():
        ts_ref[...] = ts_ref[...] + tsum


def _k3(acc01, acc23, s_nh, bias8, w1, b18, wp):
    return pl.pallas_call(
        _k3_kernel,
        grid=(N // BN3,),
        in_specs=[
            pl.BlockSpec((2, BN3, 128), lambda i: (0, i, 0)),
            pl.BlockSpec((2, BN3, 128), lambda i: (0, i, 0)),
            pl.BlockSpec((BN3, HEADS), lambda i: (i, 0)),
            pl.BlockSpec((8, D), lambda i: (0, 0)),
            pl.BlockSpec((D, SA_HID), lambda i: (0, 0)),
            pl.BlockSpec((8, SA_HID), lambda i: (0, 0)),
            pl.BlockSpec((D, OUT), lambda i: (0, 0)),
        ],
        out_specs=[
            pl.BlockSpec((BN3, OUT), lambda i: (i, 0)),
            pl.BlockSpec((8, SA_HID), lambda i: (0, 0)),
        ],
        out_shape=[
            jax.ShapeDtypeStruct((N, OUT), jnp.float32),
            jax.ShapeDtypeStruct((8, SA_HID), jnp.float32),
        ],
    )(acc01, acc23, s_nh, bias8, w1, b18, wp)


def _k4_kernel(y0_ref, y1_ref, bb_ref, bp_ref, o_ref):
    o_ref[...] = jax.nn.sigmoid(
        y0_ref[...] * bb_ref[0] + y1_ref[...] * bb_ref[1] + bp_ref[0:1, :])


def _k4(y0, y1, bb, bp8):
    return pl.pallas_call(
        _k4_kernel,
        grid=(1,),
        in_specs=[
            pl.BlockSpec((N, OUT), lambda i: (0, 0)),
            pl.BlockSpec((N, OUT), lambda i: (0, 0)),
            pl.BlockSpec(memory_space=pltpu.SMEM),
            pl.BlockSpec((8, OUT), lambda i: (0, 0)),
        ],
        out_specs=pl.BlockSpec((N, OUT), lambda i: (0, 0)),
        out_shape=jax.ShapeDtypeStruct((N, OUT), jnp.float32),
    )(y0, y1, bb, bp8)


def _attn_mat(a):
    eye = jnp.eye(HEADS, dtype=jnp.float32)
    return (eye[:, None, :] * a[:, :, None]).reshape(D, HEADS)


def kernel(h, adj, edge_index0, edge_index1, fc_W0, attn_l0, attn_r0, bias0,
           fc_W1, attn_l1, attn_r1, bias1, sa_W1, sa_b1, sa_W2, Wp, bp):
    h2 = _adj_h(adj, h)

    edge_phase = _make_edge_phase()
    agg0 = _make_aggregate(0)
    agg1 = _make_aggregate(1)

    ys = []
    ws = []
    for ei, W, al, ar, bias in (
            (edge_index0, fc_W0, attn_l0, attn_r0, bias0),
            (edge_index1, fc_W1, attn_l1, attn_r1, bias1)):
        f0, f1, f2, f3, el, er = _feat_el_er(h2, W, _attn_mat(al),
                                             _attn_mat(ar))
        el_t = el.T.reshape(-1)
        er_t = er.T.reshape(-1)
        srcc, dstlc, eec, s_t, counts = edge_phase(ei[0], ei[1], el_t, er_t)
        acc01 = agg0(f0, f1, srcc, dstlc, eec, counts)
        acc23 = agg1(f2, f3, srcc, dstlc, eec, counts)
        s_nh = s_t.reshape(HEADS, NPAD)[:, :N].T
        bias8 = jnp.broadcast_to(bias[None, :], (8, D))
        b18 = jnp.broadcast_to(sa_b1[None, :], (8, SA_HID))
        y, ts = _k3(acc01, acc23, s_nh, bias8, sa_W1, b18, Wp)
        ys.append(y)
        ws.append(jnp.dot(ts[0], sa_W2[:, 0]) / N)

    beta = jax.nn.softmax(jnp.stack(ws))
    bp8 = jnp.broadcast_to(bp[None, :], (8, OUT))
    return _k4(ys[0], ys[1], beta, bp8)
